# baseline (speedup 1.0000x reference)
import functools
import math

import jax
import jax.numpy as jnp
from jax.experimental import pallas as pl
from jax.experimental.pallas import tpu as pltpu

_TILE = 2048
_KEY_LO = -1065353220
_KEY_HI = 1065353216


def _sortable(f):
    b = jax.lax.bitcast_convert_type(f, jnp.int32)
    return jnp.where(b >= 0, b, jnp.bitwise_xor(b, jnp.int32(0x7FFFFFFF)))


def _fused_kernel(x_hbm, wxz_ref, wxh_ref, bxz_ref, bhz_ref, bxh_ref,
                  bhh_ref, pwr_ref, l1w_ref, l1b_ref, l2w_ref, l2b_ref,
                  out_ref, h_scr, xb0, xb1, sem0, sem1, *, n, k_keep, tiles):
    f32 = jnp.float32
    nrm = jnp.sqrt(jnp.sum(pwr_ref[:] * pwr_ref[:]))
    bz = bxz_ref[:] + bhz_ref[:]
    bh = bxh_ref[:] + bhh_ref[:]
    bufs = (xb0, xb1)
    sems = (sem0, sem1)

    def start_copy(i):
        a, b = tiles[i]
        cp = pltpu.make_async_copy(
            x_hbm.at[pl.ds(a, b - a), :], bufs[i % 2].at[0:b - a, :],
            sems[i % 2])
        cp.start()
        return cp

    copies = [start_copy(0)]
    st_pieces = []
    for i, (a, b) in enumerate(tiles):
        t = b - a
        copies[i].wait()
        if i + 1 < len(tiles):
            copies.append(start_copy(i + 1))
        xt = bufs[i % 2][0:t, :]
        z = jax.nn.sigmoid(
            jax.lax.dot_general(xt, wxz_ref[:], (((1,), (0,)), ((), ())),
                                preferred_element_type=f32, precision=jax.lax.Precision.HIGHEST) + bz)
        ht = jnp.tanh(
            jax.lax.dot_general(xt, wxh_ref[:], (((1,), (0,)), ((), ())),
                                preferred_element_type=f32, precision=jax.lax.Precision.HIGHEST) + bh)
        h = jnp.maximum((1.0 - z) * ht, 1e-20)
        h_scr[a:b, :] = h
        st = jnp.tanh(
            jax.lax.dot_general(pwr_ref[:], h, (((1,), (1,)), ((), ())),
                                preferred_element_type=f32, precision=jax.lax.Precision.HIGHEST) / nrm)
        st_pieces.append(st)

    s_t = jnp.concatenate(st_pieces, axis=1)
    iota_t = jax.lax.broadcasted_iota(jnp.int32, (1, n), 1)
    keys_t = _sortable(s_t)

    ukey = keys_t ^ jnp.int32(-2147483648)
    ib4 = 4 * ((max(n - 1, 1).bit_length() + 3) // 4)
    inv_t = jnp.int32((1 << ib4) - 1) - iota_t
    jio = jax.lax.broadcasted_iota(jnp.int32, (16, 1), 0)

    def radix_level(dig, definite, active, k_rem):
        digm = jnp.where(active, dig, jnp.int32(-1))
        ge = digm >= jio
        suffix = jnp.sum(ge.astype(jnp.float32), axis=1, keepdims=True)
        c = jnp.sum((suffix >= k_rem).astype(jnp.float32))
        jstar = c.astype(jnp.int32) - 1
        s_above = jnp.sum(jnp.where(jio == jstar + 1, suffix, 0.0))
        definite = definite | (digm > jstar)
        return definite, digm == jstar, k_rem - s_above

    active = jnp.ones((1, n), dtype=jnp.bool_)
    definite = jnp.zeros((1, n), dtype=jnp.bool_)
    k_rem = jnp.float32(k_keep)
    for lv in range(8):
        dig = jax.lax.shift_right_logical(ukey, 28 - 4 * lv) & 15
        definite, active, k_rem = radix_level(dig, definite, active, k_rem)

    for lv in range(ib4 // 4):
        dig = jax.lax.shift_right_logical(inv_t, ib4 - 4 - 4 * lv) & 15
        definite, active, k_rem = radix_level(dig, definite, active, k_rem)
    sel_t = definite | active

    w_t = jnp.where(sel_t, s_t, 0.0)
    q_t = jnp.where(sel_t, s_t, -1e30)
    qcol = jnp.reshape(q_t, (n, 1))
    h_all = h_scr[:, :]
    valm = h_all * qcol
    gmax = jnp.max(valm, axis=0, keepdims=True)
    gsum = jax.lax.dot_general(w_t, h_all, (((1,), (0,)), ((), ())),
                               preferred_element_type=f32, precision=jax.lax.Precision.HIGHEST)

    gmean = gsum * (1.0 / float(k_keep))
    cat = jnp.concatenate([gmax, gmean], axis=1)
    o1 = jnp.maximum(
        jax.lax.dot_general(cat, l1w_ref[:], (((1,), (0,)), ((), ())),
                            preferred_element_type=f32, precision=jax.lax.Precision.HIGHEST) + l1b_ref[:], 0.0)
    o2 = jax.lax.dot_general(o1, l2w_ref[:], (((1,), (0,)), ((), ())),
                             preferred_element_type=f32, precision=jax.lax.Precision.HIGHEST) + l2b_ref[:]
    out_ref[:] = o2


def kernel(x, edge_index, edge_attr, batch, W_xz, b_xz, W_hz, b_hz, W_xr,
           b_xr, W_hr, b_hr, W_xh, b_xh, W_hh, b_hh, pool_w, lin1_W, lin1_b,
           lin2_W, lin2_b):
    n, lookback = x.shape
    dim = W_xz.shape[1]
    out_d = lin2_W.shape[1]
    k_keep = int(math.ceil(0.8 * n))
    bounds = list(range(0, n, _TILE)) + [n]
    tiles = tuple(zip(bounds[:-1], bounds[1:]))

    body = functools.partial(_fused_kernel, n=n, k_keep=k_keep, tiles=tiles)
    res = pl.pallas_call(
        body,
        out_shape=jax.ShapeDtypeStruct((1, out_d), jnp.float32),
        in_specs=[pl.BlockSpec(memory_space=pltpu.MemorySpace.HBM)] +
                 [pl.BlockSpec(memory_space=pltpu.MemorySpace.VMEM)] * 11,
        scratch_shapes=[
            pltpu.VMEM((n, dim), jnp.float32),
            pltpu.VMEM((_TILE, lookback), jnp.float32),
            pltpu.VMEM((_TILE, lookback), jnp.float32),
            pltpu.SemaphoreType.DMA,
            pltpu.SemaphoreType.DMA,
        ],
        compiler_params=pltpu.CompilerParams(
            vmem_limit_bytes=100 * 1024 * 1024),
    )(x, W_xz, W_xh, b_xz.reshape(1, dim), b_hz.reshape(1, dim),
      b_xh.reshape(1, dim), b_hh.reshape(1, dim), pool_w.reshape(1, dim),
      lin1_W, lin1_b.reshape(1, dim), lin2_W, lin2_b.reshape(1, out_d))
    return res
